# Initial kernel scaffold; baseline (speedup 1.0000x reference)
import jax
import jax.numpy as jnp
from jax.experimental import pallas as pl


def kernel(feat, coord, batch, W1, b1, gamma, beta, W2, b2):
    raise NotImplementedError("write your pallas kernel here")


# speedup vs baseline: 157.1475x; 157.1475x over previous
import functools

import jax
import jax.numpy as jnp
from jax import lax
from jax.experimental import pallas as pl
from jax.experimental.pallas import tpu as pltpu
from jax.experimental.pallas import tpu_sc as plsc

N = 4096
K = 16
NB = 8
NC = 2
NS = 16
NW = NC * NS
QPW = N // NW
CHUNKS = N // 16


def _rsqrt16(x):
    i = plsc.bitcast(x, jnp.int32)
    i = jnp.int32(0x5F3759DF) - lax.shift_right_logical(i, 1)
    y = plsc.bitcast(i, jnp.float32)
    for _ in range(3):
        y = y * (jnp.float32(1.5) - jnp.float32(0.5) * x * y * y)
    return y


def _round_bf16(x):
    u = plsc.bitcast(x, jnp.int32)
    lsb = lax.shift_right_logical(u, 16) & jnp.int32(1)
    u = (u + jnp.int32(0x7FFF) + lsb) & jnp.int32(-65536)
    return plsc.bitcast(u, jnp.float32)


def _sc_body(cx_h, cy_h, cz_h, b_h, s_h, e_h, lin_h, dens_h,
             cxv, cyv, czv, bxv, byv, bzv, sqv, bv, sv, ev,
             ab, bb, cb, db, eb, fb, mdb, linb, densb):
    wid = lax.axis_index("s") * NC + lax.axis_index("c")
    q0 = wid * QPW

    pltpu.sync_copy(cx_h, cxv.at[pl.ds(0, N)])
    pltpu.sync_copy(cy_h, cyv.at[pl.ds(0, N)])
    pltpu.sync_copy(cz_h, czv.at[pl.ds(0, N)])
    pltpu.sync_copy(b_h, bv.at[pl.ds(0, N)])
    pltpu.sync_copy(s_h, sv)
    pltpu.sync_copy(e_h, ev)

    lane0 = lax.iota(jnp.int32, 16) == 0

    def _sget_f(ref, i):
        return ref[pl.ds(i, 16)][0]

    def _sget_i(ref, i):
        return ref[pl.ds(i, 16)][0]

    def _sput(ref, i, val):
        plsc.store_scatter(ref, [jnp.full((16,), i, jnp.int32)],
                           jnp.full((16,), val, jnp.float32), mask=lane0)

    def sq_body(ci, _):
        base = ci * 16
        x = cxv[pl.ds(base, 16)]
        y = cyv[pl.ds(base, 16)]
        z = czv[pl.ds(base, 16)]
        sqv[pl.ds(base, 16)] = x * x + y * y + z * z
        bxv[pl.ds(base, 16)] = _round_bf16(x)
        byv[pl.ds(base, 16)] = _round_bf16(y)
        bzv[pl.ds(base, 16)] = _round_bf16(z)
        return 0

    lax.fori_loop(0, CHUNKS, sq_body, 0)

    inv15 = jnp.float32(1.0 / (K - 1))
    inv16 = jnp.float32(1.0 / K)

    def q_body(j, _):
        qi = q0 + j
        qx = _sget_f(bxv, qi)
        qy = _sget_f(byv, qi)
        qz = _sget_f(bzv, qi)
        qsq = _sget_f(sqv, qi)
        qb = _sget_i(bv, qi)
        seg_s = _sget_i(sv, qb)
        seg_e = _sget_i(ev, qb)
        c_lo = seg_s // 16
        c_hi = (seg_e + 15) // 16

        def c_body(ci, carry):
            rk, rv = carry
            base = ci * 16
            sqc = sqv[pl.ds(base, 16)]
            xc = bxv[pl.ds(base, 16)]
            yc = byv[pl.ds(base, 16)]
            zc = bzv[pl.ds(base, 16)]
            bc = bv[pl.ds(base, 16)]
            d2 = (qsq + sqc) - jnp.float32(2.0) * (qx * xc + qy * yc + qz * zc)
            idxv = base + lax.iota(jnp.int32, 16)
            valid = (bc == qb) & (idxv != qi)
            dm = jnp.where(valid, d2, jnp.float32(jnp.inf))
            sk, svv = plsc.sort_key_val(dm, idxv)
            rsk = lax.rev(sk, (0,))
            rsv = lax.rev(svv, (0,))
            take = rk <= rsk
            mk = jnp.where(take, rk, rsk)
            mv = jnp.where(take, rv, rsv)
            nk, nv = plsc.sort_key_val(mk, mv)
            return nk, nv

        rk0 = jnp.full((16,), jnp.inf, jnp.float32)
        rv0 = jnp.zeros((16,), jnp.int32)
        rk, rv = lax.fori_loop(c_lo, c_hi, c_body, (rk0, rv0))

        d2c = jnp.maximum(rk, jnp.float32(1e-12))
        dist = d2c * _rsqrt16(d2c)
        _sput(mdb, j, jnp.sum(dist) * inv16)

        nx = plsc.load_gather(cxv, [rv])
        ny = plsc.load_gather(cyv, [rv])
        nz = plsc.load_gather(czv, [rv])
        mx = jnp.sum(nx) * inv16
        my = jnp.sum(ny) * inv16
        mz = jnp.sum(nz) * inv16
        dx = nx - mx
        dy = ny - my
        dz = nz - mz
        _sput(ab, j, jnp.sum(dx * dx) * inv15)
        _sput(bb, j, jnp.sum(dy * dy) * inv15)
        _sput(cb, j, jnp.sum(dz * dz) * inv15)
        _sput(db, j, jnp.sum(dx * dy) * inv15)
        _sput(eb, j, jnp.sum(dx * dz) * inv15)
        _sput(fb, j, jnp.sum(dy * dz) * inv15)
        return 0

    lax.fori_loop(0, QPW, q_body, 0)

    def p_body(t, _):
        base = t * 16
        a = ab[pl.ds(base, 16)]
        b = bb[pl.ds(base, 16)]
        c = cb[pl.ds(base, 16)]
        d = db[pl.ds(base, 16)]
        e = eb[pl.ds(base, 16)]
        f = fb[pl.ds(base, 16)]
        md = mdb[pl.ds(base, 16)]
        c2 = a + b + c
        c1 = a * b + b * c + c * a - d * d - e * e - f * f
        c0 = a * (b * c - f * f) - d * (d * c - f * e) + e * (d * f - b * e)
        lam = c2
        lo = c2 * jnp.float32(1.0 / 3.0)
        for _ in range(40):
            p = ((lam - c2) * lam + c1) * lam - c0
            pp = (jnp.float32(3.0) * lam - jnp.float32(2.0) * c2) * lam + c1
            bad = pp == jnp.float32(0.0)
            pp_safe = jnp.where(bad, jnp.float32(1.0), pp)
            lam_new = jnp.where(bad, lam, lam - p / pp_safe)
            lam = jnp.minimum(lam, jnp.maximum(lam_new, lo))
        linb[pl.ds(base, 16)] = ((jnp.float32(2.0) * lam - c2)
                                 / (c2 + jnp.float32(1e-6)))
        densb[pl.ds(base, 16)] = jnp.float32(1.0) / (md + jnp.float32(1e-6))
        return 0

    lax.fori_loop(0, QPW // 16, p_body, 0)

    pltpu.sync_copy(linb, lin_h.at[pl.ds(q0, QPW)])
    pltpu.sync_copy(densb, dens_h.at[pl.ds(q0, QPW)])


def _sc_geometry(cx, cy, cz, batch, starts, ends):
    mesh = plsc.VectorSubcoreMesh(core_axis_name="c", subcore_axis_name="s",
                                  num_cores=NC, num_subcores=NS)
    f32 = jnp.float32
    return pl.kernel(
        _sc_body,
        out_type=(jax.ShapeDtypeStruct((N,), f32),
                  jax.ShapeDtypeStruct((N,), f32)),
        mesh=mesh,
        scratch_types=[
            pltpu.VMEM((N + 16,), f32),
            pltpu.VMEM((N + 16,), f32),
            pltpu.VMEM((N + 16,), f32),
            pltpu.VMEM((N + 16,), f32),
            pltpu.VMEM((N + 16,), f32),
            pltpu.VMEM((N + 16,), f32),
            pltpu.VMEM((N + 16,), f32),
            pltpu.VMEM((N + 16,), jnp.int32),
            pltpu.VMEM((32,), jnp.int32),
            pltpu.VMEM((32,), jnp.int32),
            pltpu.VMEM((QPW,), f32),
            pltpu.VMEM((QPW,), f32),
            pltpu.VMEM((QPW,), f32),
            pltpu.VMEM((QPW,), f32),
            pltpu.VMEM((QPW,), f32),
            pltpu.VMEM((QPW,), f32),
            pltpu.VMEM((QPW,), f32),
            pltpu.VMEM((QPW,), f32),
            pltpu.VMEM((QPW,), f32),
        ],
        compiler_params=pltpu.CompilerParams(needs_layout_passes=False),
    )(cx, cy, cz, batch, starts, ends)


def _tc_body(feat_ref, w1t_ref, b1_ref, g_ref, be_ref, w2_ref, b2_ref,
             lin_ref, dens_ref, o0_ref, o2_ref):
    h = jnp.dot(feat_ref[...], w1t_ref[...],
                preferred_element_type=jnp.float32) + b1_ref[...]
    mu = jnp.mean(h, axis=0, keepdims=True)
    var = jnp.mean((h - mu) ** 2, axis=0, keepdims=True)
    h = (h - mu) / jnp.sqrt(var + jnp.float32(1e-5)) * g_ref[...] + be_ref[...]
    h = jnp.maximum(h, jnp.float32(0.0))
    l0 = jnp.sum(h * w2_ref[0:1, :], axis=1, keepdims=True) + b2_ref[0:1, 0:1]
    l1 = jnp.sum(h * w2_ref[1:2, :], axis=1, keepdims=True) + b2_ref[0:1, 1:2]
    l2 = jnp.sum(h * w2_ref[2:3, :], axis=1, keepdims=True) + b2_ref[0:1, 2:3]
    m = jnp.maximum(jnp.maximum(l0, l1), l2)
    e0 = jnp.exp(l0 - m)
    e1 = jnp.exp(l1 - m)
    e2 = jnp.exp(l2 - m)
    es = e0 + e1 + e2
    p0 = e0 / es
    p1 = e1 / es
    p2 = e2 / es
    lin = lin_ref[...]
    dens = dens_ref[...]
    third = jnp.float32(1.0 / 3.0)
    tower = (dens * jnp.float32(2.0) + p0) * third
    backg = (jnp.maximum(jnp.float32(1.0) - lin, jnp.float32(1.0) - dens)
             + p1) * third
    line = (lin * jnp.float32(2.0) + p2) * third
    eps = jnp.float32(1e-6)
    o0_ref[...] = (tower * jnp.float32(0.1) + backg * jnp.float32(0.5)
                   + line * jnp.float32(0.2) + eps)
    o2_ref[...] = (tower * jnp.float32(0.1) + backg * jnp.float32(0.5)
                   + line * jnp.float32(5.0) + eps)


def _tc_judge(feat, w1t, b1p, gp, bep, w2p, b2p, lin, dens):
    f32 = jnp.float32
    return pl.pallas_call(
        _tc_body,
        out_shape=(jax.ShapeDtypeStruct((N, 1), f32),
                   jax.ShapeDtypeStruct((N, 1), f32)),
    )(feat, w1t, b1p, gp, bep, w2p, b2p, lin, dens)


def kernel(feat, coord, batch, W1, b1, gamma, beta, W2, b2):
    f32 = jnp.float32
    coord = coord.astype(f32)
    cx = coord[:, 0]
    cy = coord[:, 1]
    cz = coord[:, 2]
    batch = batch.astype(jnp.int32)
    ar = jnp.arange(NB, dtype=jnp.int32)
    starts = jnp.searchsorted(batch, ar, side="left").astype(jnp.int32)
    ends = jnp.searchsorted(batch, ar, side="right").astype(jnp.int32)
    starts = jnp.pad(starts, (0, 32 - NB))
    ends = jnp.pad(ends, (0, 32 - NB))

    lin, dens = _sc_geometry(cx, cy, cz, batch, starts, ends)

    H = 128
    w1t = jnp.zeros((256, H), f32).at[:, :64].set(W1.T.astype(f32))
    b1p = jnp.zeros((1, H), f32).at[0, :64].set(b1.astype(f32))
    gp = jnp.zeros((1, H), f32).at[0, :64].set(gamma.astype(f32))
    bep = jnp.zeros((1, H), f32).at[0, :64].set(beta.astype(f32))
    w2p = jnp.zeros((3, H), f32).at[:, :64].set(W2.astype(f32))
    b2p = jnp.zeros((1, H), f32).at[0, :3].set(b2.astype(f32))

    o01, o2 = _tc_judge(feat.astype(f32), w1t, b1p, gp, bep, w2p, b2p,
                        lin.reshape(N, 1), dens.reshape(N, 1))
    return jnp.concatenate([o01, o01, o2], axis=1)


# speedup vs baseline: 203.1580x; 1.2928x over previous
import functools

import jax
import jax.numpy as jnp
from jax import lax
from jax.experimental import pallas as pl
from jax.experimental.pallas import tpu as pltpu
from jax.experimental.pallas import tpu_sc as plsc

N = 4096
K = 16
NB = 8
NC = 2
NS = 16
NW = NC * NS
QPW = N // NW
CHUNKS = N // 16


def _rsqrt16(x):
    i = plsc.bitcast(x, jnp.int32)
    i = jnp.int32(0x5F3759DF) - lax.shift_right_logical(i, 1)
    y = plsc.bitcast(i, jnp.float32)
    for _ in range(3):
        y = y * (jnp.float32(1.5) - jnp.float32(0.5) * x * y * y)
    return y


def _round_bf16(x):
    u = plsc.bitcast(x, jnp.int32)
    lsb = lax.shift_right_logical(u, 16) & jnp.int32(1)
    u = (u + jnp.int32(0x7FFF) + lsb) & jnp.int32(-65536)
    return plsc.bitcast(u, jnp.float32)


def _sc_body(cf_h, b_h, s_h, lin_h, dens_h,
             cfv, cxv, cyv, czv, bxv, byv, bzv, sqv, bv, sv,
             ab, bb, cb, db, eb, fb, mdb, linb, densb):
    wid = lax.axis_index("s") * NC + lax.axis_index("c")
    q0 = wid * QPW

    pltpu.sync_copy(cf_h, cfv)
    pltpu.sync_copy(b_h, bv.at[pl.ds(0, N)])
    pltpu.sync_copy(s_h, sv)

    lane0 = lax.iota(jnp.int32, 16) == 0
    iota3 = lax.iota(jnp.int32, 16) * 3

    def _sget_f(ref, i):
        return ref[pl.ds(i, 16)][0]

    def _sget_i(ref, i):
        return ref[pl.ds(i, 16)][0]

    def _sput(ref, i, val):
        plsc.store_scatter(ref, [jnp.full((16,), i, jnp.int32)],
                           jnp.full((16,), val, jnp.float32), mask=lane0)

    def sq_body(ci, _):
        base = ci * 16
        i3 = base * 3 + iota3
        x = plsc.load_gather(cfv, [i3])
        y = plsc.load_gather(cfv, [i3 + 1])
        z = plsc.load_gather(cfv, [i3 + 2])
        cxv[pl.ds(base, 16)] = x
        cyv[pl.ds(base, 16)] = y
        czv[pl.ds(base, 16)] = z
        sqv[pl.ds(base, 16)] = x * x + y * y + z * z
        bxv[pl.ds(base, 16)] = _round_bf16(x)
        byv[pl.ds(base, 16)] = _round_bf16(y)
        bzv[pl.ds(base, 16)] = _round_bf16(z)
        return 0

    lax.fori_loop(0, CHUNKS, sq_body, 0)

    inv15 = jnp.float32(1.0 / (K - 1))
    inv16 = jnp.float32(1.0 / K)

    def q_body(j, _):
        qi = q0 + j
        qx = _sget_f(bxv, qi)
        qy = _sget_f(byv, qi)
        qz = _sget_f(bzv, qi)
        qsq = _sget_f(sqv, qi)
        qb = _sget_i(bv, qi)
        seg_s = _sget_i(sv, qb)
        seg_e = _sget_i(sv, qb + 1)
        c_lo = seg_s // 16
        c_hi = (seg_e + 15) // 16

        def c_body(ci, carry):
            rk, rv = carry
            base = ci * 16
            sqc = sqv[pl.ds(base, 16)]
            xc = bxv[pl.ds(base, 16)]
            yc = byv[pl.ds(base, 16)]
            zc = bzv[pl.ds(base, 16)]
            bc = bv[pl.ds(base, 16)]
            d2 = (qsq + sqc) - jnp.float32(2.0) * (qx * xc + qy * yc + qz * zc)
            idxv = base + lax.iota(jnp.int32, 16)
            valid = (bc == qb) & (idxv != qi)
            dm = jnp.where(valid, d2, jnp.float32(jnp.inf))
            sk, svv = plsc.sort_key_val(dm, idxv)
            rsk = lax.rev(sk, (0,))
            rsv = lax.rev(svv, (0,))
            take = rk <= rsk
            mk = jnp.where(take, rk, rsk)
            mv = jnp.where(take, rv, rsv)
            nk, nv = plsc.sort_key_val(mk, mv)
            return nk, nv

        rk0 = jnp.full((16,), jnp.inf, jnp.float32)
        rv0 = jnp.zeros((16,), jnp.int32)
        rk, rv = lax.fori_loop(c_lo, c_hi, c_body, (rk0, rv0))

        d2c = jnp.maximum(rk, jnp.float32(1e-12))
        dist = d2c * _rsqrt16(d2c)
        _sput(mdb, j, jnp.sum(dist) * inv16)

        nx = plsc.load_gather(cxv, [rv])
        ny = plsc.load_gather(cyv, [rv])
        nz = plsc.load_gather(czv, [rv])
        mx = jnp.sum(nx) * inv16
        my = jnp.sum(ny) * inv16
        mz = jnp.sum(nz) * inv16
        dx = nx - mx
        dy = ny - my
        dz = nz - mz
        _sput(ab, j, jnp.sum(dx * dx) * inv15)
        _sput(bb, j, jnp.sum(dy * dy) * inv15)
        _sput(cb, j, jnp.sum(dz * dz) * inv15)
        _sput(db, j, jnp.sum(dx * dy) * inv15)
        _sput(eb, j, jnp.sum(dx * dz) * inv15)
        _sput(fb, j, jnp.sum(dy * dz) * inv15)
        return 0

    lax.fori_loop(0, QPW, q_body, 0)

    def p_body(t, _):
        base = t * 16
        a = ab[pl.ds(base, 16)]
        b = bb[pl.ds(base, 16)]
        c = cb[pl.ds(base, 16)]
        d = db[pl.ds(base, 16)]
        e = eb[pl.ds(base, 16)]
        f = fb[pl.ds(base, 16)]
        md = mdb[pl.ds(base, 16)]
        c2 = a + b + c
        c1 = a * b + b * c + c * a - d * d - e * e - f * f
        c0 = a * (b * c - f * f) - d * (d * c - f * e) + e * (d * f - b * e)
        lam = c2
        lo = c2 * jnp.float32(1.0 / 3.0)
        for _ in range(40):
            p = ((lam - c2) * lam + c1) * lam - c0
            pp = (jnp.float32(3.0) * lam - jnp.float32(2.0) * c2) * lam + c1
            bad = pp == jnp.float32(0.0)
            pp_safe = jnp.where(bad, jnp.float32(1.0), pp)
            lam_new = jnp.where(bad, lam, lam - p / pp_safe)
            lam = jnp.minimum(lam, jnp.maximum(lam_new, lo))
        linb[pl.ds(base, 16)] = ((jnp.float32(2.0) * lam - c2)
                                 / (c2 + jnp.float32(1e-6)))
        densb[pl.ds(base, 16)] = jnp.float32(1.0) / (md + jnp.float32(1e-6))
        return 0

    lax.fori_loop(0, QPW // 16, p_body, 0)

    pltpu.sync_copy(linb, lin_h.at[pl.ds(q0, QPW)])
    pltpu.sync_copy(densb, dens_h.at[pl.ds(q0, QPW)])


def _sc_geometry(coordf, batch, starts):
    mesh = plsc.VectorSubcoreMesh(core_axis_name="c", subcore_axis_name="s",
                                  num_cores=NC, num_subcores=NS)
    f32 = jnp.float32
    return pl.kernel(
        _sc_body,
        out_type=(jax.ShapeDtypeStruct((N,), f32),
                  jax.ShapeDtypeStruct((N,), f32)),
        mesh=mesh,
        scratch_types=[
            pltpu.VMEM((3 * N,), f32),
            pltpu.VMEM((N + 16,), f32),
            pltpu.VMEM((N + 16,), f32),
            pltpu.VMEM((N + 16,), f32),
            pltpu.VMEM((N + 16,), f32),
            pltpu.VMEM((N + 16,), f32),
            pltpu.VMEM((N + 16,), f32),
            pltpu.VMEM((N + 16,), f32),
            pltpu.VMEM((N + 16,), jnp.int32),
            pltpu.VMEM((32,), jnp.int32),
            pltpu.VMEM((QPW,), f32),
            pltpu.VMEM((QPW,), f32),
            pltpu.VMEM((QPW,), f32),
            pltpu.VMEM((QPW,), f32),
            pltpu.VMEM((QPW,), f32),
            pltpu.VMEM((QPW,), f32),
            pltpu.VMEM((QPW,), f32),
            pltpu.VMEM((QPW,), f32),
            pltpu.VMEM((QPW,), f32),
        ],
        compiler_params=pltpu.CompilerParams(needs_layout_passes=False),
    )(coordf, batch, starts)


def _mlp_body(feat_ref, w1t_ref, b1_ref, g_ref, be_ref, w2_ref, b2_ref,
              p0_ref, p1_ref, p2_ref):
    h = jnp.dot(feat_ref[...], w1t_ref[...],
                preferred_element_type=jnp.float32) + b1_ref[...]
    mu = jnp.mean(h, axis=0, keepdims=True)
    var = jnp.mean((h - mu) ** 2, axis=0, keepdims=True)
    h = (h - mu) / jnp.sqrt(var + jnp.float32(1e-5)) * g_ref[...] + be_ref[...]
    h = jnp.maximum(h, jnp.float32(0.0))
    l0 = jnp.sum(h * w2_ref[0:1, :], axis=1, keepdims=True) + b2_ref[0:1, 0:1]
    l1 = jnp.sum(h * w2_ref[1:2, :], axis=1, keepdims=True) + b2_ref[0:1, 1:2]
    l2 = jnp.sum(h * w2_ref[2:3, :], axis=1, keepdims=True) + b2_ref[0:1, 2:3]
    m = jnp.maximum(jnp.maximum(l0, l1), l2)
    e0 = jnp.exp(l0 - m)
    e1 = jnp.exp(l1 - m)
    e2 = jnp.exp(l2 - m)
    es = e0 + e1 + e2
    p0_ref[...] = e0 / es
    p1_ref[...] = e1 / es
    p2_ref[...] = e2 / es


def _tc_mlp(feat, w1t, b1p, gp, bep, w2p, b2p):
    f32 = jnp.float32
    return pl.pallas_call(
        _mlp_body,
        out_shape=(jax.ShapeDtypeStruct((N, 1), f32),
                   jax.ShapeDtypeStruct((N, 1), f32),
                   jax.ShapeDtypeStruct((N, 1), f32)),
    )(feat, w1t, b1p, gp, bep, w2p, b2p)


def _combine_body(lin_ref, dens_ref, p0_ref, p1_ref, p2_ref, o_ref):
    lin = lin_ref[...]
    dens = dens_ref[...]
    third = jnp.float32(1.0 / 3.0)
    tower = (dens * jnp.float32(2.0) + p0_ref[...]) * third
    backg = (jnp.maximum(jnp.float32(1.0) - lin, jnp.float32(1.0) - dens)
             + p1_ref[...]) * third
    line = (lin * jnp.float32(2.0) + p2_ref[...]) * third
    eps = jnp.float32(1e-6)
    c01 = (tower * jnp.float32(0.1) + backg * jnp.float32(0.5)
           + line * jnp.float32(0.2) + eps)
    c2 = (tower * jnp.float32(0.1) + backg * jnp.float32(0.5)
          + line * jnp.float32(5.0) + eps)
    o_ref[:, 0:1] = c01
    o_ref[:, 1:2] = c01
    o_ref[:, 2:3] = c2


def _tc_combine(lin, dens, p0, p1, p2):
    return pl.pallas_call(
        _combine_body,
        out_shape=jax.ShapeDtypeStruct((N, 3), jnp.float32),
    )(lin, dens, p0, p1, p2)


def kernel(feat, coord, batch, W1, b1, gamma, beta, W2, b2):
    f32 = jnp.float32
    coord = coord.astype(f32)
    batch = batch.astype(jnp.int32)
    ar = jnp.arange(NB + 1, dtype=jnp.int32)
    starts = jnp.sum(batch[None, :] < ar[:, None], axis=1).astype(jnp.int32)
    starts = jnp.pad(starts, (0, 32 - NB - 1))

    lin, dens = _sc_geometry(coord.reshape(-1), batch, starts)

    H = 128
    w1t = jnp.zeros((256, H), f32).at[:, :64].set(W1.T.astype(f32))
    b1p = jnp.zeros((1, H), f32).at[0, :64].set(b1.astype(f32))
    gp = jnp.zeros((1, H), f32).at[0, :64].set(gamma.astype(f32))
    bep = jnp.zeros((1, H), f32).at[0, :64].set(beta.astype(f32))
    w2p = jnp.zeros((3, H), f32).at[:, :64].set(W2.astype(f32))
    b2p = jnp.zeros((1, H), f32).at[0, :3].set(b2.astype(f32))

    p0, p1, p2 = _tc_mlp(feat.astype(f32), w1t, b1p, gp, bep, w2p, b2p)
    return _tc_combine(lin.reshape(N, 1), dens.reshape(N, 1), p0, p1, p2)


# speedup vs baseline: 227.8864x; 1.1217x over previous
import functools

import jax
import jax.numpy as jnp
from jax import lax
from jax.experimental import pallas as pl
from jax.experimental.pallas import tpu as pltpu
from jax.experimental.pallas import tpu_sc as plsc

N = 4096
K = 16
NB = 8
NC = 2
NS = 16
NW = NC * NS
QPW = N // NW
CHUNKS = N // 16


def _rsqrt16(x):
    i = plsc.bitcast(x, jnp.int32)
    i = jnp.int32(0x5F3759DF) - lax.shift_right_logical(i, 1)
    y = plsc.bitcast(i, jnp.float32)
    for _ in range(3):
        y = y * (jnp.float32(1.5) - jnp.float32(0.5) * x * y * y)
    return y


def _round_bf16(x):
    u = plsc.bitcast(x, jnp.int32)
    lsb = lax.shift_right_logical(u, 16) & jnp.int32(1)
    u = (u + jnp.int32(0x7FFF) + lsb) & jnp.int32(-65536)
    return plsc.bitcast(u, jnp.float32)


def _sc_body(cf_h, b_h, s_h, lin_h, dens_h,
             cfv, cxv, cyv, czv, bxv, byv, bzv, sqv, bv, sv,
             ab, bb, cb, db, eb, fb, mdb, linb, densb):
    wid = lax.axis_index("s") * NC + lax.axis_index("c")
    q0 = wid * QPW

    pltpu.sync_copy(cf_h, cfv)
    pltpu.sync_copy(b_h, bv.at[pl.ds(0, N)])
    pltpu.sync_copy(s_h, sv)

    lane0 = lax.iota(jnp.int32, 16) == 0
    iota3 = lax.iota(jnp.int32, 16) * 3

    def _sget_f(ref, i):
        return ref[pl.ds(i, 16)][0]

    def _sget_i(ref, i):
        return ref[pl.ds(i, 16)][0]

    def _sput(ref, i, val):
        plsc.store_scatter(ref, [jnp.full((16,), i, jnp.int32)],
                           jnp.full((16,), val, jnp.float32), mask=lane0)

    def sq_body(ci, _):
        base = ci * 16
        i3 = base * 3 + iota3
        x = plsc.load_gather(cfv, [i3])
        y = plsc.load_gather(cfv, [i3 + 1])
        z = plsc.load_gather(cfv, [i3 + 2])
        cxv[pl.ds(base, 16)] = x
        cyv[pl.ds(base, 16)] = y
        czv[pl.ds(base, 16)] = z
        sqv[pl.ds(base, 16)] = x * x + y * y + z * z
        bxv[pl.ds(base, 16)] = _round_bf16(x)
        byv[pl.ds(base, 16)] = _round_bf16(y)
        bzv[pl.ds(base, 16)] = _round_bf16(z)
        return 0

    lax.fori_loop(0, CHUNKS, sq_body, 0)

    inv15 = jnp.float32(1.0 / (K - 1))
    inv16 = jnp.float32(1.0 / K)

    def q_body(j, _):
        qi = q0 + j
        qx = _sget_f(bxv, qi)
        qy = _sget_f(byv, qi)
        qz = _sget_f(bzv, qi)
        qsq = _sget_f(sqv, qi)
        qb = _sget_i(bv, qi)
        seg_s = _sget_i(sv, qb)
        seg_e = _sget_i(sv, qb + 1)
        c_lo = seg_s // 16
        c_hi = (seg_e + 15) // 16

        def c_body(ci, carry):
            rk, rv = carry
            base = ci * 16
            sqc = sqv[pl.ds(base, 16)]
            xc = bxv[pl.ds(base, 16)]
            yc = byv[pl.ds(base, 16)]
            zc = bzv[pl.ds(base, 16)]
            bc = bv[pl.ds(base, 16)]
            d2 = (qsq + sqc) - jnp.float32(2.0) * (qx * xc + qy * yc + qz * zc)
            idxv = base + lax.iota(jnp.int32, 16)
            valid = (bc == qb) & (idxv != qi)
            dm = jnp.where(valid, d2, jnp.float32(jnp.inf))
            sk, svv = plsc.sort_key_val(dm, idxv)
            rsk = lax.rev(sk, (0,))
            rsv = lax.rev(svv, (0,))
            take = rk <= rsk
            mk = jnp.where(take, rk, rsk)
            mv = jnp.where(take, rv, rsv)
            nk, nv = plsc.sort_key_val(mk, mv)
            return nk, nv

        rk0 = jnp.full((16,), jnp.inf, jnp.float32)
        rv0 = jnp.zeros((16,), jnp.int32)
        rk, rv = lax.fori_loop(c_lo, c_hi, c_body, (rk0, rv0))

        d2c = jnp.maximum(rk, jnp.float32(1e-12))
        dist = d2c * _rsqrt16(d2c)
        _sput(mdb, j, jnp.sum(dist) * inv16)

        nx = plsc.load_gather(cxv, [rv])
        ny = plsc.load_gather(cyv, [rv])
        nz = plsc.load_gather(czv, [rv])
        mx = jnp.sum(nx) * inv16
        my = jnp.sum(ny) * inv16
        mz = jnp.sum(nz) * inv16
        dx = nx - mx
        dy = ny - my
        dz = nz - mz
        _sput(ab, j, jnp.sum(dx * dx) * inv15)
        _sput(bb, j, jnp.sum(dy * dy) * inv15)
        _sput(cb, j, jnp.sum(dz * dz) * inv15)
        _sput(db, j, jnp.sum(dx * dy) * inv15)
        _sput(eb, j, jnp.sum(dx * dz) * inv15)
        _sput(fb, j, jnp.sum(dy * dz) * inv15)
        return 0

    lax.fori_loop(0, QPW, q_body, 0)

    def p_body(t, _):
        base = t * 16
        a = ab[pl.ds(base, 16)]
        b = bb[pl.ds(base, 16)]
        c = cb[pl.ds(base, 16)]
        d = db[pl.ds(base, 16)]
        e = eb[pl.ds(base, 16)]
        f = fb[pl.ds(base, 16)]
        md = mdb[pl.ds(base, 16)]
        c2 = a + b + c
        c1 = a * b + b * c + c * a - d * d - e * e - f * f
        c0 = a * (b * c - f * f) - d * (d * c - f * e) + e * (d * f - b * e)
        lam = c2
        lo = c2 * jnp.float32(1.0 / 3.0)
        for _ in range(40):
            p = ((lam - c2) * lam + c1) * lam - c0
            pp = (jnp.float32(3.0) * lam - jnp.float32(2.0) * c2) * lam + c1
            bad = pp == jnp.float32(0.0)
            pp_safe = jnp.where(bad, jnp.float32(1.0), pp)
            lam_new = jnp.where(bad, lam, lam - p / pp_safe)
            lam = jnp.minimum(lam, jnp.maximum(lam_new, lo))
        linb[pl.ds(base, 16)] = ((jnp.float32(2.0) * lam - c2)
                                 / (c2 + jnp.float32(1e-6)))
        densb[pl.ds(base, 16)] = jnp.float32(1.0) / (md + jnp.float32(1e-6))
        return 0

    lax.fori_loop(0, QPW // 16, p_body, 0)

    pltpu.sync_copy(linb, lin_h.at[pl.ds(q0, QPW)])
    pltpu.sync_copy(densb, dens_h.at[pl.ds(q0, QPW)])


def _sc_geometry(coordf, batch, starts):
    mesh = plsc.VectorSubcoreMesh(core_axis_name="c", subcore_axis_name="s",
                                  num_cores=NC, num_subcores=NS)
    f32 = jnp.float32
    return pl.kernel(
        _sc_body,
        out_type=(jax.ShapeDtypeStruct((N,), f32),
                  jax.ShapeDtypeStruct((N,), f32)),
        mesh=mesh,
        scratch_types=[
            pltpu.VMEM((3 * N,), f32),
            pltpu.VMEM((N + 16,), f32),
            pltpu.VMEM((N + 16,), f32),
            pltpu.VMEM((N + 16,), f32),
            pltpu.VMEM((N + 16,), f32),
            pltpu.VMEM((N + 16,), f32),
            pltpu.VMEM((N + 16,), f32),
            pltpu.VMEM((N + 16,), f32),
            pltpu.VMEM((N + 16,), jnp.int32),
            pltpu.VMEM((32,), jnp.int32),
            pltpu.VMEM((QPW,), f32),
            pltpu.VMEM((QPW,), f32),
            pltpu.VMEM((QPW,), f32),
            pltpu.VMEM((QPW,), f32),
            pltpu.VMEM((QPW,), f32),
            pltpu.VMEM((QPW,), f32),
            pltpu.VMEM((QPW,), f32),
            pltpu.VMEM((QPW,), f32),
            pltpu.VMEM((QPW,), f32),
        ],
        compiler_params=pltpu.CompilerParams(needs_layout_passes=False),
        cost_estimate=pl.CostEstimate(flops=400_000_000, transcendentals=0,
                                      bytes_accessed=2_000_000),
    )(coordf, batch, starts)


def _mlp_body(feat_ref, w1t_ref, b1_ref, g_ref, be_ref, w2_ref, b2_ref,
              p0_ref, p1_ref, p2_ref):
    h = jnp.dot(feat_ref[...], w1t_ref[...],
                preferred_element_type=jnp.float32) + b1_ref[...]
    mu = jnp.mean(h, axis=0, keepdims=True)
    var = jnp.mean((h - mu) ** 2, axis=0, keepdims=True)
    h = (h - mu) / jnp.sqrt(var + jnp.float32(1e-5)) * g_ref[...] + be_ref[...]
    h = jnp.maximum(h, jnp.float32(0.0))
    l0 = jnp.sum(h * w2_ref[0:1, :], axis=1) + b2_ref[0, 0]
    l1 = jnp.sum(h * w2_ref[1:2, :], axis=1) + b2_ref[0, 1]
    l2 = jnp.sum(h * w2_ref[2:3, :], axis=1) + b2_ref[0, 2]
    m = jnp.maximum(jnp.maximum(l0, l1), l2)
    e0 = jnp.exp(l0 - m)
    e1 = jnp.exp(l1 - m)
    e2 = jnp.exp(l2 - m)
    es = e0 + e1 + e2
    p0_ref[...] = e0 / es
    p1_ref[...] = e1 / es
    p2_ref[...] = e2 / es


def _tc_mlp(feat, w1t, b1p, gp, bep, w2p, b2p):
    f32 = jnp.float32
    return pl.pallas_call(
        _mlp_body,
        out_shape=(jax.ShapeDtypeStruct((N,), f32),
                   jax.ShapeDtypeStruct((N,), f32),
                   jax.ShapeDtypeStruct((N,), f32)),
    )(feat, w1t, b1p, gp, bep, w2p, b2p)


def _combine_body(lin_ref, dens_ref, p0_ref, p1_ref, p2_ref, o01_ref, o2_ref):
    lin = lin_ref[...]
    dens = dens_ref[...]
    third = jnp.float32(1.0 / 3.0)
    tower = (dens * jnp.float32(2.0) + p0_ref[...]) * third
    backg = (jnp.maximum(jnp.float32(1.0) - lin, jnp.float32(1.0) - dens)
             + p1_ref[...]) * third
    line = (lin * jnp.float32(2.0) + p2_ref[...]) * third
    eps = jnp.float32(1e-6)
    o01_ref[...] = (tower * jnp.float32(0.1) + backg * jnp.float32(0.5)
                    + line * jnp.float32(0.2) + eps)
    o2_ref[...] = (tower * jnp.float32(0.1) + backg * jnp.float32(0.5)
                   + line * jnp.float32(5.0) + eps)


def _tc_combine(lin, dens, p0, p1, p2):
    return pl.pallas_call(
        _combine_body,
        out_shape=(jax.ShapeDtypeStruct((N,), jnp.float32),
                   jax.ShapeDtypeStruct((N,), jnp.float32)),
    )(lin, dens, p0, p1, p2)


def kernel(feat, coord, batch, W1, b1, gamma, beta, W2, b2):
    f32 = jnp.float32
    coord = coord.astype(f32)
    batch = batch.astype(jnp.int32)
    ar = jnp.arange(NB + 1, dtype=jnp.int32)
    starts = jnp.sum(batch[None, :] < ar[:, None], axis=1).astype(jnp.int32)
    starts = jnp.pad(starts, (0, 32 - NB - 1))

    lin, dens = _sc_geometry(coord.reshape(-1), batch, starts)

    H = 128
    w1t = jnp.zeros((256, H), f32).at[:, :64].set(W1.T.astype(f32))
    b1p = jnp.zeros((1, H), f32).at[0, :64].set(b1.astype(f32))
    gp = jnp.zeros((1, H), f32).at[0, :64].set(gamma.astype(f32))
    bep = jnp.zeros((1, H), f32).at[0, :64].set(beta.astype(f32))
    w2p = jnp.zeros((3, H), f32).at[:, :64].set(W2.astype(f32))
    b2p = jnp.zeros((1, H), f32).at[0, :3].set(b2.astype(f32))

    p0, p1, p2 = _tc_mlp(feat.astype(f32), w1t, b1p, gp, bep, w2p, b2p)
    o01, o2 = _tc_combine(lin, dens, p0, p1, p2)
    return jnp.stack([o01, o01, o2], axis=1)


# speedup vs baseline: 228.6595x; 1.0034x over previous
import functools

import jax
import jax.numpy as jnp
from jax import lax
from jax.experimental import pallas as pl
from jax.experimental.pallas import tpu as pltpu
from jax.experimental.pallas import tpu_sc as plsc

N = 4096
K = 16
NB = 8
NC = 2
NS = 16
NW = NC * NS
QPW = N // NW
CHUNKS = N // 16


def _rsqrt16(x):
    i = plsc.bitcast(x, jnp.int32)
    i = jnp.int32(0x5F3759DF) - lax.shift_right_logical(i, 1)
    y = plsc.bitcast(i, jnp.float32)
    for _ in range(3):
        y = y * (jnp.float32(1.5) - jnp.float32(0.5) * x * y * y)
    return y


def _round_bf16(x):
    u = plsc.bitcast(x, jnp.int32)
    lsb = lax.shift_right_logical(u, 16) & jnp.int32(1)
    u = (u + jnp.int32(0x7FFF) + lsb) & jnp.int32(-65536)
    return plsc.bitcast(u, jnp.float32)


def _sc_body(cf_h, b_h, s_h, lin_h, dens_h,
             cfv, cxv, cyv, czv, bxv, byv, bzv, sqv, bv, sv,
             ab, bb, cb, db, eb, fb, mdb, linb, densb):
    wid = lax.axis_index("s") * NC + lax.axis_index("c")
    q0 = wid * QPW

    pltpu.sync_copy(cf_h, cfv)
    pltpu.sync_copy(b_h, bv.at[pl.ds(0, N)])
    pltpu.sync_copy(s_h, sv)

    lane0 = lax.iota(jnp.int32, 16) == 0
    iota3 = lax.iota(jnp.int32, 16) * 3

    def _sget_f(ref, i):
        return ref[pl.ds(i, 16)][0]

    def _sget_i(ref, i):
        return ref[pl.ds(i, 16)][0]

    def _sput(ref, i, val):
        plsc.store_scatter(ref, [jnp.full((16,), i, jnp.int32)],
                           jnp.full((16,), val, jnp.float32), mask=lane0)

    def sq_body(ci, _):
        base = ci * 16
        i3 = base * 3 + iota3
        x = plsc.load_gather(cfv, [i3])
        y = plsc.load_gather(cfv, [i3 + 1])
        z = plsc.load_gather(cfv, [i3 + 2])
        cxv[pl.ds(base, 16)] = x
        cyv[pl.ds(base, 16)] = y
        czv[pl.ds(base, 16)] = z
        sqv[pl.ds(base, 16)] = x * x + y * y + z * z
        bxv[pl.ds(base, 16)] = _round_bf16(x)
        byv[pl.ds(base, 16)] = _round_bf16(y)
        bzv[pl.ds(base, 16)] = _round_bf16(z)
        return 0

    lax.fori_loop(0, CHUNKS, sq_body, 0)

    inv15 = jnp.float32(1.0 / (K - 1))
    inv16 = jnp.float32(1.0 / K)

    def q_body(j, _):
        qi = q0 + j
        qx = _sget_f(bxv, qi)
        qy = _sget_f(byv, qi)
        qz = _sget_f(bzv, qi)
        qsq = _sget_f(sqv, qi)
        qb = _sget_i(bv, qi)
        seg_s = _sget_i(sv, qb)
        seg_e = _sget_i(sv, qb + 1)
        c_lo = seg_s // 16
        c_hi = (seg_e + 15) // 16

        def c_body(ci, carry):
            rk, rv = carry
            base = ci * 16
            sqc = sqv[pl.ds(base, 16)]
            xc = bxv[pl.ds(base, 16)]
            yc = byv[pl.ds(base, 16)]
            zc = bzv[pl.ds(base, 16)]
            bc = bv[pl.ds(base, 16)]
            d2 = (qsq + sqc) - jnp.float32(2.0) * (qx * xc + qy * yc + qz * zc)
            idxv = base + lax.iota(jnp.int32, 16)
            valid = (bc == qb) & (idxv != qi)
            dm = jnp.where(valid, d2, jnp.float32(jnp.inf))
            sk, svv = plsc.sort_key_val(dm, idxv)
            rsk = lax.rev(sk, (0,))
            rsv = lax.rev(svv, (0,))
            take = rk <= rsk
            mk = jnp.where(take, rk, rsk)
            mv = jnp.where(take, rv, rsv)
            nk, nv = plsc.sort_key_val(mk, mv)
            return nk, nv

        rk0 = jnp.full((16,), jnp.inf, jnp.float32)
        rv0 = jnp.zeros((16,), jnp.int32)
        rk, rv = lax.fori_loop(c_lo, c_hi, c_body, (rk0, rv0))

        d2c = jnp.maximum(rk, jnp.float32(1e-12))
        dist = d2c * _rsqrt16(d2c)
        _sput(mdb, j, jnp.sum(dist) * inv16)

        nx = plsc.load_gather(cxv, [rv])
        ny = plsc.load_gather(cyv, [rv])
        nz = plsc.load_gather(czv, [rv])
        mx = jnp.sum(nx) * inv16
        my = jnp.sum(ny) * inv16
        mz = jnp.sum(nz) * inv16
        dx = nx - mx
        dy = ny - my
        dz = nz - mz
        _sput(ab, j, jnp.sum(dx * dx) * inv15)
        _sput(bb, j, jnp.sum(dy * dy) * inv15)
        _sput(cb, j, jnp.sum(dz * dz) * inv15)
        _sput(db, j, jnp.sum(dx * dy) * inv15)
        _sput(eb, j, jnp.sum(dx * dz) * inv15)
        _sput(fb, j, jnp.sum(dy * dz) * inv15)
        return 0

    lax.fori_loop(0, QPW, q_body, 0)

    def p_body(t, _):
        base = t * 16
        a = ab[pl.ds(base, 16)]
        b = bb[pl.ds(base, 16)]
        c = cb[pl.ds(base, 16)]
        d = db[pl.ds(base, 16)]
        e = eb[pl.ds(base, 16)]
        f = fb[pl.ds(base, 16)]
        md = mdb[pl.ds(base, 16)]
        c2 = a + b + c
        c1 = a * b + b * c + c * a - d * d - e * e - f * f
        c0 = a * (b * c - f * f) - d * (d * c - f * e) + e * (d * f - b * e)
        lam = c2
        lo = c2 * jnp.float32(1.0 / 3.0)
        for _ in range(40):
            p = ((lam - c2) * lam + c1) * lam - c0
            pp = (jnp.float32(3.0) * lam - jnp.float32(2.0) * c2) * lam + c1
            bad = pp == jnp.float32(0.0)
            pp_safe = jnp.where(bad, jnp.float32(1.0), pp)
            lam_new = jnp.where(bad, lam, lam - p / pp_safe)
            lam = jnp.minimum(lam, jnp.maximum(lam_new, lo))
        linb[pl.ds(base, 16)] = ((jnp.float32(2.0) * lam - c2)
                                 / (c2 + jnp.float32(1e-6)))
        densb[pl.ds(base, 16)] = jnp.float32(1.0) / (md + jnp.float32(1e-6))
        return 0

    lax.fori_loop(0, QPW // 16, p_body, 0)

    pltpu.sync_copy(linb, lin_h.at[pl.ds(q0, QPW)])
    pltpu.sync_copy(densb, dens_h.at[pl.ds(q0, QPW)])


def _sc_geometry(coordf, batch, starts):
    mesh = plsc.VectorSubcoreMesh(core_axis_name="c", subcore_axis_name="s",
                                  num_cores=NC, num_subcores=NS)
    f32 = jnp.float32
    return pl.kernel(
        _sc_body,
        out_type=(jax.ShapeDtypeStruct((N,), f32),
                  jax.ShapeDtypeStruct((N,), f32)),
        mesh=mesh,
        scratch_types=[
            pltpu.VMEM((3 * N,), f32),
            pltpu.VMEM((N + 16,), f32),
            pltpu.VMEM((N + 16,), f32),
            pltpu.VMEM((N + 16,), f32),
            pltpu.VMEM((N + 16,), f32),
            pltpu.VMEM((N + 16,), f32),
            pltpu.VMEM((N + 16,), f32),
            pltpu.VMEM((N + 16,), f32),
            pltpu.VMEM((N + 16,), jnp.int32),
            pltpu.VMEM((32,), jnp.int32),
            pltpu.VMEM((QPW,), f32),
            pltpu.VMEM((QPW,), f32),
            pltpu.VMEM((QPW,), f32),
            pltpu.VMEM((QPW,), f32),
            pltpu.VMEM((QPW,), f32),
            pltpu.VMEM((QPW,), f32),
            pltpu.VMEM((QPW,), f32),
            pltpu.VMEM((QPW,), f32),
            pltpu.VMEM((QPW,), f32),
        ],
        compiler_params=pltpu.CompilerParams(needs_layout_passes=False),
        cost_estimate=pl.CostEstimate(flops=400_000_000, transcendentals=0,
                                      bytes_accessed=2_000_000),
    )(coordf, batch, starts)


def _mlp_body(feat_ref, w1_ref, b1_ref, g_ref, be_ref, w2_ref, b2_ref,
              p0_ref, p1_ref, p2_ref):
    h = jax.lax.dot_general(feat_ref[...], w1_ref[...],
                            (((1,), (1,)), ((), ())),
                            preferred_element_type=jnp.float32)
    h = h + b1_ref[...]
    mu = jnp.mean(h, axis=0, keepdims=True)
    var = jnp.mean((h - mu) ** 2, axis=0, keepdims=True)
    h = (h - mu) / jnp.sqrt(var + jnp.float32(1e-5)) * g_ref[...] + be_ref[...]
    h = jnp.maximum(h, jnp.float32(0.0))
    l0 = jnp.sum(h * w2_ref[0:1, :], axis=1) + b2_ref[0:1]
    l1 = jnp.sum(h * w2_ref[1:2, :], axis=1) + b2_ref[1:2]
    l2 = jnp.sum(h * w2_ref[2:3, :], axis=1) + b2_ref[2:3]
    m = jnp.maximum(jnp.maximum(l0, l1), l2)
    e0 = jnp.exp(l0 - m)
    e1 = jnp.exp(l1 - m)
    e2 = jnp.exp(l2 - m)
    es = e0 + e1 + e2
    p0_ref[...] = e0 / es
    p1_ref[...] = e1 / es
    p2_ref[...] = e2 / es


def _tc_mlp(feat, w1, b1p, gp, bep, w2, b2p):
    f32 = jnp.float32
    return pl.pallas_call(
        _mlp_body,
        out_shape=(jax.ShapeDtypeStruct((N,), f32),
                   jax.ShapeDtypeStruct((N,), f32),
                   jax.ShapeDtypeStruct((N,), f32)),
    )(feat, w1, b1p, gp, bep, w2, b2p)


def _combine_body(lin_ref, dens_ref, p0_ref, p1_ref, p2_ref, o01_ref, o2_ref):
    lin = lin_ref[...]
    dens = dens_ref[...]
    third = jnp.float32(1.0 / 3.0)
    tower = (dens * jnp.float32(2.0) + p0_ref[...]) * third
    backg = (jnp.maximum(jnp.float32(1.0) - lin, jnp.float32(1.0) - dens)
             + p1_ref[...]) * third
    line = (lin * jnp.float32(2.0) + p2_ref[...]) * third
    eps = jnp.float32(1e-6)
    o01_ref[...] = (tower * jnp.float32(0.1) + backg * jnp.float32(0.5)
                    + line * jnp.float32(0.2) + eps)
    o2_ref[...] = (tower * jnp.float32(0.1) + backg * jnp.float32(0.5)
                   + line * jnp.float32(5.0) + eps)


def _tc_combine(lin, dens, p0, p1, p2):
    return pl.pallas_call(
        _combine_body,
        out_shape=(jax.ShapeDtypeStruct((N,), jnp.float32),
                   jax.ShapeDtypeStruct((N,), jnp.float32)),
    )(lin, dens, p0, p1, p2)


def kernel(feat, coord, batch, W1, b1, gamma, beta, W2, b2):
    f32 = jnp.float32
    coord = coord.astype(f32)
    batch = batch.astype(jnp.int32)
    ar = jnp.arange(NB + 1, dtype=jnp.int32)
    starts = jnp.sum(batch[None, :] < ar[:, None], axis=1).astype(jnp.int32)
    starts = jnp.pad(starts, (0, 32 - NB - 1))

    lin, dens = _sc_geometry(coord.reshape(-1), batch, starts)

    p0, p1, p2 = _tc_mlp(feat.astype(f32), W1.astype(f32), b1.astype(f32),
                         gamma.astype(f32), beta.astype(f32), W2.astype(f32),
                         b2.astype(f32))
    o01, o2 = _tc_combine(lin, dens, p0, p1, p2)
    return jnp.stack([o01, o01, o2], axis=1)


# speedup vs baseline: 237.9137x; 1.0405x over previous
import functools

import jax
import jax.numpy as jnp
from jax import lax
from jax.experimental import pallas as pl
from jax.experimental.pallas import tpu as pltpu
from jax.experimental.pallas import tpu_sc as plsc

N = 4096
K = 16
NB = 8
NC = 2
NS = 16
NW = NC * NS
QPW = N // NW
CHUNKS = N // 16


def _rsqrt16(x):
    i = plsc.bitcast(x, jnp.int32)
    i = jnp.int32(0x5F3759DF) - lax.shift_right_logical(i, 1)
    y = plsc.bitcast(i, jnp.float32)
    for _ in range(3):
        y = y * (jnp.float32(1.5) - jnp.float32(0.5) * x * y * y)
    return y


def _round_bf16(x):
    u = plsc.bitcast(x, jnp.int32)
    lsb = lax.shift_right_logical(u, 16) & jnp.int32(1)
    u = (u + jnp.int32(0x7FFF) + lsb) & jnp.int32(-65536)
    return plsc.bitcast(u, jnp.float32)


def _sc_body(cf_h, b_h, s_h, lin_h, dens_h,
             cfv, cxv, cyv, czv, bxv, byv, bzv, sqv, bv, sv,
             ab, bb, cb, db, eb, fb, mdb, linb, densb):
    wid = lax.axis_index("s") * NC + lax.axis_index("c")
    q0 = wid * QPW

    pltpu.sync_copy(cf_h, cfv)
    pltpu.sync_copy(b_h, bv.at[pl.ds(0, N)])
    pltpu.sync_copy(s_h, sv)

    lane0 = lax.iota(jnp.int32, 16) == 0
    iota3 = lax.iota(jnp.int32, 16) * 3
    lane15 = jnp.full((16,), 15, jnp.int32)

    def _splat(ref, i):
        return plsc.load_gather(ref, [jnp.full((16,), i, jnp.int32)])

    def _ssum(x):
        return jnp.take_along_axis(jnp.cumsum(x), lane15, axis=0,
                                   mode="promise_in_bounds")

    def _sput(ref, i, val):
        plsc.store_scatter(ref, [jnp.full((16,), i, jnp.int32)],
                           val, mask=lane0)

    def sq_body(ci, _):
        base = ci * 16
        i3 = base * 3 + iota3
        x = plsc.load_gather(cfv, [i3])
        y = plsc.load_gather(cfv, [i3 + 1])
        z = plsc.load_gather(cfv, [i3 + 2])
        cxv[pl.ds(base, 16)] = x
        cyv[pl.ds(base, 16)] = y
        czv[pl.ds(base, 16)] = z
        sqv[pl.ds(base, 16)] = x * x + y * y + z * z
        bxv[pl.ds(base, 16)] = _round_bf16(x)
        byv[pl.ds(base, 16)] = _round_bf16(y)
        bzv[pl.ds(base, 16)] = _round_bf16(z)
        return 0

    lax.fori_loop(0, CHUNKS, sq_body, 0)

    inv15 = jnp.float32(1.0 / (K - 1))
    inv16 = jnp.float32(1.0 / K)

    def q_body(j, _):
        qi = q0 + j
        qx = _splat(bxv, qi)
        qy = _splat(byv, qi)
        qz = _splat(bzv, qi)
        qsq = _splat(sqv, qi)
        qbv = _splat(bv, qi)
        qiv = jnp.full((16,), qi, jnp.int32)
        seg_s = plsc.load_gather(sv, [qbv])[0]
        seg_e = plsc.load_gather(sv, [qbv + 1])[0]
        c_lo = seg_s // 16
        c_hi = (seg_e + 15) // 16

        def c_body(ci, carry):
            rk, rv = carry
            base = ci * 16
            sqc = sqv[pl.ds(base, 16)]
            xc = bxv[pl.ds(base, 16)]
            yc = byv[pl.ds(base, 16)]
            zc = bzv[pl.ds(base, 16)]
            bc = bv[pl.ds(base, 16)]
            d2 = (qsq + sqc) - jnp.float32(2.0) * (qx * xc + qy * yc + qz * zc)
            idxv = base + lax.iota(jnp.int32, 16)
            valid = (bc == qbv) & (idxv != qiv)
            dm = jnp.where(valid, d2, jnp.float32(jnp.inf))
            sk, svv = plsc.sort_key_val(dm, idxv, descending=True)
            take = rk <= sk
            mk = jnp.where(take, rk, sk)
            mv = jnp.where(take, rv, svv)
            nk, nv = plsc.sort_key_val(mk, mv)
            return nk, nv

        rk0 = jnp.full((16,), jnp.inf, jnp.float32)
        rv0 = jnp.zeros((16,), jnp.int32)
        rk, rv = lax.fori_loop(c_lo, c_hi, c_body, (rk0, rv0))

        d2c = jnp.maximum(rk, jnp.float32(1e-12))
        dist = d2c * _rsqrt16(d2c)
        _sput(mdb, j, _ssum(dist) * inv16)

        nx = plsc.load_gather(cxv, [rv])
        ny = plsc.load_gather(cyv, [rv])
        nz = plsc.load_gather(czv, [rv])
        mx = _ssum(nx) * inv16
        my = _ssum(ny) * inv16
        mz = _ssum(nz) * inv16
        dx = nx - mx
        dy = ny - my
        dz = nz - mz
        _sput(ab, j, _ssum(dx * dx) * inv15)
        _sput(bb, j, _ssum(dy * dy) * inv15)
        _sput(cb, j, _ssum(dz * dz) * inv15)
        _sput(db, j, _ssum(dx * dy) * inv15)
        _sput(eb, j, _ssum(dx * dz) * inv15)
        _sput(fb, j, _ssum(dy * dz) * inv15)
        return 0

    lax.fori_loop(0, QPW, q_body, 0)

    def p_body(t, _):
        base = t * 16
        a = ab[pl.ds(base, 16)]
        b = bb[pl.ds(base, 16)]
        c = cb[pl.ds(base, 16)]
        d = db[pl.ds(base, 16)]
        e = eb[pl.ds(base, 16)]
        f = fb[pl.ds(base, 16)]
        md = mdb[pl.ds(base, 16)]
        c2 = a + b + c
        c1 = a * b + b * c + c * a - d * d - e * e - f * f
        c0 = a * (b * c - f * f) - d * (d * c - f * e) + e * (d * f - b * e)
        lam = c2
        lo = c2 * jnp.float32(1.0 / 3.0)
        for _ in range(40):
            p = ((lam - c2) * lam + c1) * lam - c0
            pp = (jnp.float32(3.0) * lam - jnp.float32(2.0) * c2) * lam + c1
            bad = pp == jnp.float32(0.0)
            pp_safe = jnp.where(bad, jnp.float32(1.0), pp)
            lam_new = jnp.where(bad, lam, lam - p / pp_safe)
            lam = jnp.minimum(lam, jnp.maximum(lam_new, lo))
        linb[pl.ds(base, 16)] = ((jnp.float32(2.0) * lam - c2)
                                 / (c2 + jnp.float32(1e-6)))
        densb[pl.ds(base, 16)] = jnp.float32(1.0) / (md + jnp.float32(1e-6))
        return 0

    lax.fori_loop(0, QPW // 16, p_body, 0)

    pltpu.sync_copy(linb, lin_h.at[pl.ds(q0, QPW)])
    pltpu.sync_copy(densb, dens_h.at[pl.ds(q0, QPW)])


def _sc_geometry(coordf, batch, starts):
    mesh = plsc.VectorSubcoreMesh(core_axis_name="c", subcore_axis_name="s",
                                  num_cores=NC, num_subcores=NS)
    f32 = jnp.float32
    return pl.kernel(
        _sc_body,
        out_type=(jax.ShapeDtypeStruct((N,), f32),
                  jax.ShapeDtypeStruct((N,), f32)),
        mesh=mesh,
        scratch_types=[
            pltpu.VMEM((3 * N,), f32),
            pltpu.VMEM((N + 16,), f32),
            pltpu.VMEM((N + 16,), f32),
            pltpu.VMEM((N + 16,), f32),
            pltpu.VMEM((N + 16,), f32),
            pltpu.VMEM((N + 16,), f32),
            pltpu.VMEM((N + 16,), f32),
            pltpu.VMEM((N + 16,), f32),
            pltpu.VMEM((N + 16,), jnp.int32),
            pltpu.VMEM((32,), jnp.int32),
            pltpu.VMEM((QPW,), f32),
            pltpu.VMEM((QPW,), f32),
            pltpu.VMEM((QPW,), f32),
            pltpu.VMEM((QPW,), f32),
            pltpu.VMEM((QPW,), f32),
            pltpu.VMEM((QPW,), f32),
            pltpu.VMEM((QPW,), f32),
            pltpu.VMEM((QPW,), f32),
            pltpu.VMEM((QPW,), f32),
        ],
        compiler_params=pltpu.CompilerParams(needs_layout_passes=False),
        cost_estimate=pl.CostEstimate(flops=400_000_000, transcendentals=0,
                                      bytes_accessed=2_000_000),
    )(coordf, batch, starts)


def _mlp_body(feat_ref, w1_ref, b1_ref, g_ref, be_ref, w2_ref, b2_ref,
              p0_ref, p1_ref, p2_ref):
    h = jax.lax.dot_general(feat_ref[...], w1_ref[...],
                            (((1,), (1,)), ((), ())),
                            preferred_element_type=jnp.float32)
    h = h + b1_ref[...]
    mu = jnp.mean(h, axis=0, keepdims=True)
    var = jnp.mean((h - mu) ** 2, axis=0, keepdims=True)
    h = (h - mu) / jnp.sqrt(var + jnp.float32(1e-5)) * g_ref[...] + be_ref[...]
    h = jnp.maximum(h, jnp.float32(0.0))
    l0 = jnp.sum(h * w2_ref[0:1, :], axis=1) + b2_ref[0:1]
    l1 = jnp.sum(h * w2_ref[1:2, :], axis=1) + b2_ref[1:2]
    l2 = jnp.sum(h * w2_ref[2:3, :], axis=1) + b2_ref[2:3]
    m = jnp.maximum(jnp.maximum(l0, l1), l2)
    e0 = jnp.exp(l0 - m)
    e1 = jnp.exp(l1 - m)
    e2 = jnp.exp(l2 - m)
    es = e0 + e1 + e2
    p0_ref[...] = e0 / es
    p1_ref[...] = e1 / es
    p2_ref[...] = e2 / es


def _tc_mlp(feat, w1, b1p, gp, bep, w2, b2p):
    f32 = jnp.float32
    return pl.pallas_call(
        _mlp_body,
        out_shape=(jax.ShapeDtypeStruct((N,), f32),
                   jax.ShapeDtypeStruct((N,), f32),
                   jax.ShapeDtypeStruct((N,), f32)),
    )(feat, w1, b1p, gp, bep, w2, b2p)


def _combine_body(lin_ref, dens_ref, p0_ref, p1_ref, p2_ref, o01_ref, o2_ref):
    lin = lin_ref[...]
    dens = dens_ref[...]
    third = jnp.float32(1.0 / 3.0)
    tower = (dens * jnp.float32(2.0) + p0_ref[...]) * third
    backg = (jnp.maximum(jnp.float32(1.0) - lin, jnp.float32(1.0) - dens)
             + p1_ref[...]) * third
    line = (lin * jnp.float32(2.0) + p2_ref[...]) * third
    eps = jnp.float32(1e-6)
    o01_ref[...] = (tower * jnp.float32(0.1) + backg * jnp.float32(0.5)
                    + line * jnp.float32(0.2) + eps)
    o2_ref[...] = (tower * jnp.float32(0.1) + backg * jnp.float32(0.5)
                   + line * jnp.float32(5.0) + eps)


def _tc_combine(lin, dens, p0, p1, p2):
    return pl.pallas_call(
        _combine_body,
        out_shape=(jax.ShapeDtypeStruct((N,), jnp.float32),
                   jax.ShapeDtypeStruct((N,), jnp.float32)),
    )(lin, dens, p0, p1, p2)


def kernel(feat, coord, batch, W1, b1, gamma, beta, W2, b2):
    f32 = jnp.float32
    coord = coord.astype(f32)
    batch = batch.astype(jnp.int32)
    ar = jnp.arange(NB + 1, dtype=jnp.int32)
    starts = jnp.sum(batch[None, :] < ar[:, None], axis=1).astype(jnp.int32)
    starts = jnp.pad(starts, (0, 32 - NB - 1))

    lin, dens = _sc_geometry(coord.reshape(-1), batch, starts)

    p0, p1, p2 = _tc_mlp(feat.astype(f32), W1.astype(f32), b1.astype(f32),
                         gamma.astype(f32), beta.astype(f32), W2.astype(f32),
                         b2.astype(f32))
    o01, o2 = _tc_combine(lin, dens, p0, p1, p2)
    return jnp.stack([o01, o01, o2], axis=1)


# speedup vs baseline: 390.8614x; 1.6429x over previous
import functools

import jax
import jax.numpy as jnp
from jax import lax
from jax.experimental import pallas as pl
from jax.experimental.pallas import tpu as pltpu
from jax.experimental.pallas import tpu_sc as plsc

N = 4096
K = 16
NB = 8
NC = 2
NS = 16
NW = NC * NS
QPW = N // NW
CHUNKS = N // 16


def _rsqrt16(x):
    i = plsc.bitcast(x, jnp.int32)
    i = jnp.int32(0x5F3759DF) - lax.shift_right_logical(i, 1)
    y = plsc.bitcast(i, jnp.float32)
    for _ in range(3):
        y = y * (jnp.float32(1.5) - jnp.float32(0.5) * x * y * y)
    return y


def _round_bf16(x):
    u = plsc.bitcast(x, jnp.int32)
    lsb = lax.shift_right_logical(u, 16) & jnp.int32(1)
    u = (u + jnp.int32(0x7FFF) + lsb) & jnp.int32(-65536)
    return plsc.bitcast(u, jnp.float32)


def _sc_body(cf_h, b_h, s_h, lin_h, dens_h,
             cfv, cxv, cyv, czv, bxv, byv, bzv, sqv, bv, sv,
             ab, bb, cb, db, eb, fb, mdb, linb, densb):
    wid = lax.axis_index("s") * NC + lax.axis_index("c")
    q0 = wid * QPW

    pltpu.sync_copy(cf_h, cfv)
    pltpu.sync_copy(b_h, bv.at[pl.ds(0, N)])
    pltpu.sync_copy(s_h, sv)

    lane0 = lax.iota(jnp.int32, 16) == 0
    iota3 = lax.iota(jnp.int32, 16) * 3
    lane15 = jnp.full((16,), 15, jnp.int32)

    def _splat(ref, i):
        return plsc.load_gather(ref, [jnp.full((16,), i, jnp.int32)])

    def _ssum(x):
        return jnp.take_along_axis(jnp.cumsum(x), lane15, axis=0,
                                   mode="promise_in_bounds")

    def _sput(ref, i, val):
        plsc.store_scatter(ref, [jnp.full((16,), i, jnp.int32)],
                           val, mask=lane0)

    def sq_body(ci, _):
        base = ci * 16
        i3 = base * 3 + iota3
        x = plsc.load_gather(cfv, [i3])
        y = plsc.load_gather(cfv, [i3 + 1])
        z = plsc.load_gather(cfv, [i3 + 2])
        cxv[pl.ds(base, 16)] = x
        cyv[pl.ds(base, 16)] = y
        czv[pl.ds(base, 16)] = z
        sqv[pl.ds(base, 16)] = x * x + y * y + z * z
        bxv[pl.ds(base, 16)] = _round_bf16(x)
        byv[pl.ds(base, 16)] = _round_bf16(y)
        bzv[pl.ds(base, 16)] = _round_bf16(z)
        return 0

    lax.fori_loop(0, CHUNKS, sq_body, 0)

    inv15 = jnp.float32(1.0 / (K - 1))
    inv16 = jnp.float32(1.0 / K)

    QG = 4

    def q_body(g, _):
        j0 = g * QG
        qi0 = q0 + j0
        qx = [_splat(bxv, qi0 + i) for i in range(QG)]
        qy = [_splat(byv, qi0 + i) for i in range(QG)]
        qz = [_splat(bzv, qi0 + i) for i in range(QG)]
        qsq = [_splat(sqv, qi0 + i) for i in range(QG)]
        qbv = [_splat(bv, qi0 + i) for i in range(QG)]
        qiv = [jnp.full((16,), qi0 + i, jnp.int32) for i in range(QG)]
        seg_s = plsc.load_gather(sv, [qbv[0]])[0]
        seg_e = plsc.load_gather(sv, [qbv[QG - 1] + 1])[0]
        c_lo = seg_s // 16
        c_hi = (seg_e + 15) // 16

        def c_body(ci, carry):
            base = ci * 16
            sqc = sqv[pl.ds(base, 16)]
            xc = bxv[pl.ds(base, 16)]
            yc = byv[pl.ds(base, 16)]
            zc = bzv[pl.ds(base, 16)]
            bc = bv[pl.ds(base, 16)]
            idxv = base + lax.iota(jnp.int32, 16)
            out = []
            for i in range(QG):
                rk, rv = carry[2 * i], carry[2 * i + 1]
                d2 = ((qsq[i] + sqc)
                      - jnp.float32(2.0) * (qx[i] * xc + qy[i] * yc
                                            + qz[i] * zc))
                valid = (bc == qbv[i]) & (idxv != qiv[i])
                dm = jnp.where(valid, d2, jnp.float32(jnp.inf))
                sk, svv = plsc.sort_key_val(dm, idxv, descending=True)
                take = rk <= sk
                mk = jnp.where(take, rk, sk)
                mv = jnp.where(take, rv, svv)
                nk, nv = plsc.sort_key_val(mk, mv)
                out += [nk, nv]
            return tuple(out)

        init = ()
        for _i in range(QG):
            init += (jnp.full((16,), jnp.inf, jnp.float32),
                     jnp.zeros((16,), jnp.int32))
        res = lax.fori_loop(c_lo, c_hi, c_body, init)

        for i in range(QG):
            j = j0 + i
            rk, rv = res[2 * i], res[2 * i + 1]
            d2c = jnp.maximum(rk, jnp.float32(1e-12))
            dist = d2c * _rsqrt16(d2c)
            _sput(mdb, j, _ssum(dist) * inv16)

            nx = plsc.load_gather(cxv, [rv])
            ny = plsc.load_gather(cyv, [rv])
            nz = plsc.load_gather(czv, [rv])
            mx = _ssum(nx) * inv16
            my = _ssum(ny) * inv16
            mz = _ssum(nz) * inv16
            dx = nx - mx
            dy = ny - my
            dz = nz - mz
            _sput(ab, j, _ssum(dx * dx) * inv15)
            _sput(bb, j, _ssum(dy * dy) * inv15)
            _sput(cb, j, _ssum(dz * dz) * inv15)
            _sput(db, j, _ssum(dx * dy) * inv15)
            _sput(eb, j, _ssum(dx * dz) * inv15)
            _sput(fb, j, _ssum(dy * dz) * inv15)
        return 0

    lax.fori_loop(0, QPW // QG, q_body, 0)

    def p_body(t, _):
        base = t * 16
        a = ab[pl.ds(base, 16)]
        b = bb[pl.ds(base, 16)]
        c = cb[pl.ds(base, 16)]
        d = db[pl.ds(base, 16)]
        e = eb[pl.ds(base, 16)]
        f = fb[pl.ds(base, 16)]
        md = mdb[pl.ds(base, 16)]
        c2 = a + b + c
        c1 = a * b + b * c + c * a - d * d - e * e - f * f
        c0 = a * (b * c - f * f) - d * (d * c - f * e) + e * (d * f - b * e)
        lam = c2
        lo = c2 * jnp.float32(1.0 / 3.0)
        for _ in range(40):
            p = ((lam - c2) * lam + c1) * lam - c0
            pp = (jnp.float32(3.0) * lam - jnp.float32(2.0) * c2) * lam + c1
            bad = pp == jnp.float32(0.0)
            pp_safe = jnp.where(bad, jnp.float32(1.0), pp)
            lam_new = jnp.where(bad, lam, lam - p / pp_safe)
            lam = jnp.minimum(lam, jnp.maximum(lam_new, lo))
        linb[pl.ds(base, 16)] = ((jnp.float32(2.0) * lam - c2)
                                 / (c2 + jnp.float32(1e-6)))
        densb[pl.ds(base, 16)] = jnp.float32(1.0) / (md + jnp.float32(1e-6))
        return 0

    lax.fori_loop(0, QPW // 16, p_body, 0)

    pltpu.sync_copy(linb, lin_h.at[pl.ds(q0, QPW)])
    pltpu.sync_copy(densb, dens_h.at[pl.ds(q0, QPW)])


def _sc_geometry(coordf, batch, starts):
    mesh = plsc.VectorSubcoreMesh(core_axis_name="c", subcore_axis_name="s",
                                  num_cores=NC, num_subcores=NS)
    f32 = jnp.float32
    return pl.kernel(
        _sc_body,
        out_type=(jax.ShapeDtypeStruct((N,), f32),
                  jax.ShapeDtypeStruct((N,), f32)),
        mesh=mesh,
        scratch_types=[
            pltpu.VMEM((3 * N,), f32),
            pltpu.VMEM((N + 16,), f32),
            pltpu.VMEM((N + 16,), f32),
            pltpu.VMEM((N + 16,), f32),
            pltpu.VMEM((N + 16,), f32),
            pltpu.VMEM((N + 16,), f32),
            pltpu.VMEM((N + 16,), f32),
            pltpu.VMEM((N + 16,), f32),
            pltpu.VMEM((N + 16,), jnp.int32),
            pltpu.VMEM((32,), jnp.int32),
            pltpu.VMEM((QPW,), f32),
            pltpu.VMEM((QPW,), f32),
            pltpu.VMEM((QPW,), f32),
            pltpu.VMEM((QPW,), f32),
            pltpu.VMEM((QPW,), f32),
            pltpu.VMEM((QPW,), f32),
            pltpu.VMEM((QPW,), f32),
            pltpu.VMEM((QPW,), f32),
            pltpu.VMEM((QPW,), f32),
        ],
        compiler_params=pltpu.CompilerParams(needs_layout_passes=False),
        cost_estimate=pl.CostEstimate(flops=400_000_000, transcendentals=0,
                                      bytes_accessed=2_000_000),
    )(coordf, batch, starts)


def _mlp_body(feat_ref, w1_ref, b1_ref, g_ref, be_ref, w2_ref, b2_ref,
              p0_ref, p1_ref, p2_ref):
    h = jax.lax.dot_general(feat_ref[...], w1_ref[...],
                            (((1,), (1,)), ((), ())),
                            preferred_element_type=jnp.float32)
    h = h + b1_ref[...]
    mu = jnp.mean(h, axis=0, keepdims=True)
    var = jnp.mean((h - mu) ** 2, axis=0, keepdims=True)
    h = (h - mu) / jnp.sqrt(var + jnp.float32(1e-5)) * g_ref[...] + be_ref[...]
    h = jnp.maximum(h, jnp.float32(0.0))
    l0 = jnp.sum(h * w2_ref[0:1, :], axis=1) + b2_ref[0:1]
    l1 = jnp.sum(h * w2_ref[1:2, :], axis=1) + b2_ref[1:2]
    l2 = jnp.sum(h * w2_ref[2:3, :], axis=1) + b2_ref[2:3]
    m = jnp.maximum(jnp.maximum(l0, l1), l2)
    e0 = jnp.exp(l0 - m)
    e1 = jnp.exp(l1 - m)
    e2 = jnp.exp(l2 - m)
    es = e0 + e1 + e2
    p0_ref[...] = e0 / es
    p1_ref[...] = e1 / es
    p2_ref[...] = e2 / es


def _tc_mlp(feat, w1, b1p, gp, bep, w2, b2p):
    f32 = jnp.float32
    return pl.pallas_call(
        _mlp_body,
        out_shape=(jax.ShapeDtypeStruct((N,), f32),
                   jax.ShapeDtypeStruct((N,), f32),
                   jax.ShapeDtypeStruct((N,), f32)),
    )(feat, w1, b1p, gp, bep, w2, b2p)


def _combine_body(lin_ref, dens_ref, p0_ref, p1_ref, p2_ref, o01_ref, o2_ref):
    lin = lin_ref[...]
    dens = dens_ref[...]
    third = jnp.float32(1.0 / 3.0)
    tower = (dens * jnp.float32(2.0) + p0_ref[...]) * third
    backg = (jnp.maximum(jnp.float32(1.0) - lin, jnp.float32(1.0) - dens)
             + p1_ref[...]) * third
    line = (lin * jnp.float32(2.0) + p2_ref[...]) * third
    eps = jnp.float32(1e-6)
    o01_ref[...] = (tower * jnp.float32(0.1) + backg * jnp.float32(0.5)
                    + line * jnp.float32(0.2) + eps)
    o2_ref[...] = (tower * jnp.float32(0.1) + backg * jnp.float32(0.5)
                   + line * jnp.float32(5.0) + eps)


def _tc_combine(lin, dens, p0, p1, p2):
    return pl.pallas_call(
        _combine_body,
        out_shape=(jax.ShapeDtypeStruct((N,), jnp.float32),
                   jax.ShapeDtypeStruct((N,), jnp.float32)),
    )(lin, dens, p0, p1, p2)


def kernel(feat, coord, batch, W1, b1, gamma, beta, W2, b2):
    f32 = jnp.float32
    coord = coord.astype(f32)
    batch = batch.astype(jnp.int32)
    ar = jnp.arange(NB + 1, dtype=jnp.int32)
    starts = jnp.sum(batch[None, :] < ar[:, None], axis=1).astype(jnp.int32)
    starts = jnp.pad(starts, (0, 32 - NB - 1))

    lin, dens = _sc_geometry(coord.reshape(-1), batch, starts)

    p0, p1, p2 = _tc_mlp(feat.astype(f32), W1.astype(f32), b1.astype(f32),
                         gamma.astype(f32), beta.astype(f32), W2.astype(f32),
                         b2.astype(f32))
    o01, o2 = _tc_combine(lin, dens, p0, p1, p2)
    return jnp.stack([o01, o01, o2], axis=1)
